# baseline (speedup 1.0000x reference)
import functools

import jax
import jax.numpy as jnp
from jax import lax
from jax.experimental import pallas as pl
from jax.experimental.pallas import tpu as pltpu
from jax.experimental.pallas import tpu_sc as plsc

NC = 2
NS = 16
L = 16
B = 16384
F = 64
NU = 100001
NM = 100000
BS = B // NS
CH = 128
NCHT = BS // CH
FIT = 2
NIT = (F // NC) // FIT


def _mf_body(user_r, movie_r, utt_r, mtt_r, but_r, bmt_r, bias_r, out_r,
             uidx, midx, uval, mval, psum, bu, bvec,
             ubufs, mbufs, ssem, gsem, bsem):
    c = lax.axis_index("c")
    s = lax.axis_index("s")
    base = s * BS

    pltpu.sync_copy(user_r.at[pl.ds(s * NCHT, NCHT)], uidx)
    pltpu.sync_copy(movie_r.at[pl.ds(s * NCHT, NCHT)], midx)
    pltpu.sync_copy(bias_r, bvec)

    @pl.when(c == 0)
    def _():
        for j in range(NCHT):
            pltpu.async_copy(
                but_r.at[uidx.at[j]], bu.at[pl.ds(j * CH, CH)], bsem)


    stage_descs = {}

    def fire_stage(it):
        buf = it % 2

        @pl.when(s == 0)
        def _():
            for half, fbase in ((0, 0), (1, F // NC)):
                @pl.when(c == half)
                def _(fbase=fbase):
                    descs = []
                    for k in range(FIT):
                        f = fbase + it * FIT + k
                        descs.append(pltpu.async_copy(
                            utt_r.at[f], ubufs[buf][k], ssem))
                        descs.append(pltpu.async_copy(
                            mtt_r.at[f], mbufs[buf][k], ssem))
                    stage_descs[(it, half)] = descs

    def wait_stage(it):
        @pl.when(s == 0)
        def _():
            for half in (0, 1):
                @pl.when(c == half)
                def _(half=half):
                    for d in stage_descs[(it, half)]:
                        d.wait()

    fire_stage(0)

    for it in range(NIT):
        buf = it % 2
        wait_stage(it)
        plsc.subcore_barrier()
        if it + 1 < NIT:
            fire_stage(it + 1)

        for k in range(FIT):
            def fire(j, carry, k=k, buf=buf):
                pltpu.async_copy(
                    ubufs[buf][k].at[uidx.at[j]],
                    uval.at[k, pl.ds(j * CH, CH)], gsem)
                pltpu.async_copy(
                    mbufs[buf][k].at[midx.at[j]],
                    mval.at[k, pl.ds(j * CH, CH)], gsem)
                return carry
            lax.fori_loop(0, NCHT, fire, 0)

        pltpu.make_async_copy(
            utt_r.at[pl.ds(0, FIT), pl.ds(0, BS)], uval, gsem).wait()
        pltpu.make_async_copy(
            utt_r.at[pl.ds(0, FIT), pl.ds(0, BS)], mval, gsem).wait()

        def fma(sl, carry, it=it):
            ds = pl.ds(sl * L, L)
            acc = uval[0, ds] * mval[0, ds]
            for k in range(1, FIT):
                acc = acc + uval[k, ds] * mval[k, ds]
            if it > 0:
                acc = acc + psum[ds]
            psum[ds] = acc
            return carry
        lax.fori_loop(0, BS // L, fma, 0)

    @pl.when(c == 0)
    def _():
        for j in range(NCHT):
            pltpu.async_copy(
                bmt_r.at[midx.at[j]], mval.at[0, pl.ds(j * CH, CH)], bsem)
        pltpu.make_async_copy(utt_r.at[0, pl.ds(0, BS)], bu, bsem).wait()
        pltpu.make_async_copy(utt_r.at[0, pl.ds(0, BS)],
                              mval.at[0], bsem).wait()
        bias_v = bvec[...]

        def badd(sl, carry):
            ds = pl.ds(sl * L, L)
            psum[ds] = psum[ds] + bu[ds] + mval[0, ds] + bias_v
            return carry
        lax.fori_loop(0, BS // L, badd, 0)

    pltpu.sync_copy(psum, out_r.at[c, pl.ds(base, BS)])


@functools.partial(
    pl.kernel,
    out_type=jax.ShapeDtypeStruct((NC, B), jnp.float32),
    mesh=plsc.VectorSubcoreMesh(
        core_axis_name="c", subcore_axis_name="s",
        num_cores=NC, num_subcores=NS),
    compiler_params=pltpu.CompilerParams(
        needs_layout_passes=False, use_tc_tiling_on_sc=True),
    scratch_types=[
        pltpu.VMEM((NCHT, CH), jnp.int32),
        pltpu.VMEM((NCHT, CH), jnp.int32),
        pltpu.VMEM((FIT, BS), jnp.float32),
        pltpu.VMEM((FIT, BS), jnp.float32),
        pltpu.VMEM((BS,), jnp.float32),
        pltpu.VMEM((BS,), jnp.float32),
        pltpu.VMEM((L,), jnp.float32),
        [[pltpu.VMEM_SHARED((NU,), jnp.float32) for _ in range(FIT)]
         for _ in range(2)],
        [[pltpu.VMEM_SHARED((NM,), jnp.float32) for _ in range(FIT)]
         for _ in range(2)],
        pltpu.SemaphoreType.DMA,
        pltpu.SemaphoreType.DMA,
        pltpu.SemaphoreType.DMA,
    ],
)
def _mf_kernel(user_r, movie_r, utt_r, mtt_r, but_r, bmt_r, bias_r, out_r,
               uidx, midx, uval, mval, psum, bu, bvec,
               ubufs, mbufs, ssem, gsem, bsem):
    _mf_body(user_r, movie_r, utt_r, mtt_r, but_r, bmt_r, bias_r, out_r,
             uidx, midx, uval, mval, psum, bu, bvec,
             ubufs, mbufs, ssem, gsem, bsem)


@jax.jit
def kernel(user, movie, user_table, movie_table, bias_user_table,
           bias_movie_table, bias):
    user2 = user.astype(jnp.int32).reshape(B // CH, CH)
    movie2 = movie.astype(jnp.int32).reshape(B // CH, CH)
    bias16 = jnp.broadcast_to(bias.astype(jnp.float32), (L,))
    parts = _mf_kernel(user2, movie2, user_table.T, movie_table.T,
                       bias_user_table.reshape(-1),
                       bias_movie_table.reshape(-1), bias16)
    return parts[0] + parts[1]


# speedup vs baseline: 1.0505x; 1.0505x over previous
import functools

import jax
import jax.numpy as jnp
from jax import lax
from jax.experimental import pallas as pl
from jax.experimental.pallas import tpu as pltpu
from jax.experimental.pallas import tpu_sc as plsc

NC = 2
NS = 16
L = 16
B = 16384
F = 64
NU = 100001
NM = 100000
BS = B // NS
CH = 128
NCHT = BS // CH
FIT = 4
NIT = (F // NC) // FIT


def _mf_body(user_r, movie_r, utt_r, mtt_r, but_r, bmt_r, bias_r, out_r,
             uidx, midx, uval, mval, psum, bu, bvec,
             ubufs, mbufs, ssem, gsem, bsem):
    c = lax.axis_index("c")
    s = lax.axis_index("s")
    base = s * BS

    pltpu.sync_copy(user_r.at[pl.ds(s * NCHT, NCHT)], uidx)
    pltpu.sync_copy(movie_r.at[pl.ds(s * NCHT, NCHT)], midx)
    pltpu.sync_copy(bias_r, bvec)

    @pl.when(c == 0)
    def _():
        for j in range(NCHT):
            pltpu.async_copy(
                but_r.at[uidx.at[j]], bu.at[pl.ds(j * CH, CH)], bsem)


    stage_descs = {}

    def fire_stage(it):
        buf = it % 2
        for sid, table_r, bufs, tag in ((0, utt_r, ubufs, "u"),
                                        (1, mtt_r, mbufs, "m")):
            @pl.when(s == sid)
            def _(table_r=table_r, bufs=bufs, tag=tag):
                for half, fbase in ((0, 0), (1, F // NC)):
                    @pl.when(c == half)
                    def _(fbase=fbase, table_r=table_r, bufs=bufs, tag=tag):
                        descs = []
                        for k in range(FIT):
                            f = fbase + it * FIT + k
                            descs.append(pltpu.async_copy(
                                table_r.at[f], bufs[buf][k], ssem))
                        stage_descs[(it, half, tag)] = descs

    def wait_stage(it):
        for sid, tag in ((0, "u"), (1, "m")):
            @pl.when(s == sid)
            def _(tag=tag):
                for half in (0, 1):
                    @pl.when(c == half)
                    def _(half=half, tag=tag):
                        for d in stage_descs[(it, half, tag)]:
                            d.wait()

    fire_stage(0)

    for it in range(NIT):
        buf = it % 2
        wait_stage(it)
        plsc.subcore_barrier()
        if it + 1 < NIT:
            fire_stage(it + 1)

        for k in range(FIT):
            def fire(j, carry, k=k, buf=buf):
                pltpu.async_copy(
                    ubufs[buf][k].at[uidx.at[j]],
                    uval.at[k, pl.ds(j * CH, CH)], gsem)
                pltpu.async_copy(
                    mbufs[buf][k].at[midx.at[j]],
                    mval.at[k, pl.ds(j * CH, CH)], gsem)
                return carry
            lax.fori_loop(0, NCHT, fire, 0)

        pltpu.make_async_copy(
            utt_r.at[pl.ds(0, FIT), pl.ds(0, BS)], uval, gsem).wait()
        pltpu.make_async_copy(
            utt_r.at[pl.ds(0, FIT), pl.ds(0, BS)], mval, gsem).wait()

        def fma(sl, carry, it=it):
            ds = pl.ds(sl * L, L)
            acc = uval[0, ds] * mval[0, ds]
            for k in range(1, FIT):
                acc = acc + uval[k, ds] * mval[k, ds]
            if it > 0:
                acc = acc + psum[ds]
            psum[ds] = acc
            return carry
        lax.fori_loop(0, BS // L, fma, 0)

    @pl.when(c == 0)
    def _():
        for j in range(NCHT):
            pltpu.async_copy(
                bmt_r.at[midx.at[j]], mval.at[0, pl.ds(j * CH, CH)], bsem)
        pltpu.make_async_copy(utt_r.at[0, pl.ds(0, BS)], bu, bsem).wait()
        pltpu.make_async_copy(utt_r.at[0, pl.ds(0, BS)],
                              mval.at[0], bsem).wait()
        bias_v = bvec[...]

        def badd(sl, carry):
            ds = pl.ds(sl * L, L)
            psum[ds] = psum[ds] + bu[ds] + mval[0, ds] + bias_v
            return carry
        lax.fori_loop(0, BS // L, badd, 0)

    pltpu.sync_copy(psum, out_r.at[c, pl.ds(base, BS)])


@functools.partial(
    pl.kernel,
    out_type=jax.ShapeDtypeStruct((NC, B), jnp.float32),
    mesh=plsc.VectorSubcoreMesh(
        core_axis_name="c", subcore_axis_name="s",
        num_cores=NC, num_subcores=NS),
    compiler_params=pltpu.CompilerParams(
        needs_layout_passes=False, use_tc_tiling_on_sc=True),
    scratch_types=[
        pltpu.VMEM((NCHT, CH), jnp.int32),
        pltpu.VMEM((NCHT, CH), jnp.int32),
        pltpu.VMEM((FIT, BS), jnp.float32),
        pltpu.VMEM((FIT, BS), jnp.float32),
        pltpu.VMEM((BS,), jnp.float32),
        pltpu.VMEM((BS,), jnp.float32),
        pltpu.VMEM((L,), jnp.float32),
        [[pltpu.VMEM_SHARED((NU,), jnp.float32) for _ in range(FIT)]
         for _ in range(2)],
        [[pltpu.VMEM_SHARED((NM,), jnp.float32) for _ in range(FIT)]
         for _ in range(2)],
        pltpu.SemaphoreType.DMA,
        pltpu.SemaphoreType.DMA,
        pltpu.SemaphoreType.DMA,
    ],
)
def _mf_kernel(user_r, movie_r, utt_r, mtt_r, but_r, bmt_r, bias_r, out_r,
               uidx, midx, uval, mval, psum, bu, bvec,
               ubufs, mbufs, ssem, gsem, bsem):
    _mf_body(user_r, movie_r, utt_r, mtt_r, but_r, bmt_r, bias_r, out_r,
             uidx, midx, uval, mval, psum, bu, bvec,
             ubufs, mbufs, ssem, gsem, bsem)


@jax.jit
def kernel(user, movie, user_table, movie_table, bias_user_table,
           bias_movie_table, bias):
    user2 = user.astype(jnp.int32).reshape(B // CH, CH)
    movie2 = movie.astype(jnp.int32).reshape(B // CH, CH)
    bias16 = jnp.broadcast_to(bias.astype(jnp.float32), (L,))
    parts = _mf_kernel(user2, movie2, user_table.T, movie_table.T,
                       bias_user_table.reshape(-1),
                       bias_movie_table.reshape(-1), bias16)
    return parts[0] + parts[1]
